# Initial kernel scaffold; baseline (speedup 1.0000x reference)
import jax
import jax.numpy as jnp
from jax.experimental import pallas as pl


def kernel(input_ids, emb_table, W, b):
    raise NotImplementedError("write your pallas kernel here")


# speedup vs baseline: 1.5317x; 1.5317x over previous
import functools

import jax
import jax.numpy as jnp
from jax import lax
from jax.experimental import pallas as pl
from jax.experimental.pallas import tpu as pltpu
from jax.experimental.pallas import tpu_sc as plsc

NC = 2
NS = 16
L = 16
NW = NC * NS
NOUT = 3
NPAD = 128
NBUF = 4


@functools.lru_cache(maxsize=None)
def _build_pool(B, T, V, D):
    assert D == 2 * L, "kernel assumes d_model == 32"
    CPR = -(-T // 128)
    assert T % CPR == 0
    CHUNK = T // CPR
    assert CHUNK % 4 == 0
    CH = CHUNK + ((-CHUNK) % 8)
    assert B % NW == 0
    RPW = B // NW
    assert NBUF % CPR == 0
    RPQ = NBUF // CPR
    assert RPW % RPQ == 0
    NQ = RPW // RPQ
    CPW = RPW * CPR

    mesh = plsc.VectorSubcoreMesh(core_axis_name="c", subcore_axis_name="s")

    @functools.partial(
        pl.kernel,
        out_type=jax.ShapeDtypeStruct((B * D,), jnp.float32),
        mesh=mesh,
        compiler_params=pltpu.CompilerParams(use_tc_tiling_on_sc=False),
        scratch_types=[
            pltpu.VMEM((CPW, CH), jnp.int32),
            pltpu.VMEM((CH, D), jnp.float32),
            pltpu.VMEM((CH, D), jnp.float32),
            pltpu.VMEM((CH, D), jnp.float32),
            pltpu.VMEM((CH, D), jnp.float32),
            pltpu.VMEM((RPW * D,), jnp.float32),
            pltpu.SemaphoreType.DMA,
            pltpu.SemaphoreType.DMA,
            pltpu.SemaphoreType.DMA,
            pltpu.SemaphoreType.DMA,
        ],
    )
    def pool_kernel(ids_hbm, table_hbm, out_hbm,
                    idx_v, bu0, bu1, bu2, bu3, pooled_v, s0, s1, s2, s3):
        bufs = (bu0, bu1, bu2, bu3)
        sems = (s0, s1, s2, s3)
        wid = lax.axis_index("s") * NC + lax.axis_index("c")
        chunk0 = wid * CPW

        pltpu.sync_copy(ids_hbm.at[pl.ds(chunk0, CPW)], idx_v)

        def issue(k, b):
            pltpu.async_copy(table_hbm.at[idx_v.at[k]], bufs[b], sems[b])

        def drain(b):
            pltpu.make_async_copy(
                table_hbm.at[pl.ds(0, CH)], bufs[b], sems[b]).wait()

        def accum_chunk(buf, a0, a1):
            def step(i, carry):
                c0, c1 = carry
                t = i * 4
                p0 = (buf[t, pl.ds(0, L)] + buf[t + 1, pl.ds(0, L)]) + (
                    buf[t + 2, pl.ds(0, L)] + buf[t + 3, pl.ds(0, L)])
                p1 = (buf[t, pl.ds(L, L)] + buf[t + 1, pl.ds(L, L)]) + (
                    buf[t + 2, pl.ds(L, L)] + buf[t + 3, pl.ds(L, L)])
                return (c0 + p0, c1 + p1)
            return lax.fori_loop(0, CHUNK // 4, step, (a0, a1))

        for b in range(NBUF):
            issue(b, b)

        zero = jnp.zeros((L,), jnp.float32)

        def quad_body(q, _):
            for half in range(RPQ):
                a0, a1 = zero, zero
                for c in range(CPR):
                    b = half * CPR + c
                    k = q * NBUF + b
                    drain(b)
                    a0, a1 = accum_chunk(bufs[b], a0, a1)

                    @pl.when(q < NQ - 1)
                    def _():
                        issue(k + NBUF, b)

                r = q * RPQ + half
                pooled_v[pl.ds(r * D, L)] = a0
                pooled_v[pl.ds(r * D + L, L)] = a1
            return 0

        lax.fori_loop(0, NQ, quad_body, 0)
        pltpu.sync_copy(pooled_v, out_hbm.at[pl.ds(wid * RPW * D, RPW * D)])

    return pool_kernel, CPR, CHUNK, CH


@functools.lru_cache(maxsize=None)
def _build_classifier(B, D):
    BM = min(B, 512)
    assert B % BM == 0

    def body(p_ref, w_ref, b_ref, o_ref):
        o_ref[...] = jnp.dot(
            p_ref[...], w_ref[...],
            preferred_element_type=jnp.float32) + b_ref[...]

    return pl.pallas_call(
        body,
        grid=(B // BM,),
        in_specs=[
            pl.BlockSpec((BM, D), lambda i: (i, 0)),
            pl.BlockSpec((D, NPAD), lambda i: (0, 0)),
            pl.BlockSpec((1, NPAD), lambda i: (0, 0)),
        ],
        out_specs=pl.BlockSpec((BM, NPAD), lambda i: (i, 0)),
        out_shape=jax.ShapeDtypeStruct((B, NPAD), jnp.float32),
    )


@jax.jit
def kernel(input_ids, emb_table, W, b):
    B, T = input_ids.shape
    V, D = emb_table.shape
    pool, CPR, CHUNK, CH = _build_pool(B, T, V, D)
    ids2 = input_ids.astype(jnp.int32).reshape(B * CPR, CHUNK)
    ids2 = jnp.pad(ids2, ((0, 0), (0, CH - CHUNK)))
    pooled = pool(ids2, emb_table).reshape(B, D)
    wpad = jnp.zeros((D, NPAD), jnp.float32)
    wpad = wpad.at[:, :NOUT].set(W.astype(jnp.float32) * (1.0 / T))
    bpad = jnp.zeros((1, NPAD), jnp.float32).at[0, :NOUT].set(
        b.astype(jnp.float32))
    logits = _build_classifier(B, D)(pooled, wpad, bpad)
    return logits[:, :NOUT]


# speedup vs baseline: 2.3330x; 1.5231x over previous
import functools

import jax
import jax.numpy as jnp
from jax import lax
from jax.experimental import pallas as pl
from jax.experimental.pallas import tpu as pltpu
from jax.experimental.pallas import tpu_sc as plsc

NC = 2
NS = 16
L = 16
NW = NC * NS
NOUT = 3
NPAD = 128
NBUF = 4


@functools.lru_cache(maxsize=None)
def _build_pool(B, T, V, D):
    assert D == 2 * L, "kernel assumes d_model == 32"
    assert T % 8 == 0 and T <= 2 * 128
    SZ = (T // 2 + ((-(T // 2)) % 8), T - (T // 2 + ((-(T // 2)) % 8)))
    OFF = (0, SZ[0])
    assert SZ[0] % 4 == 0 and SZ[1] % 4 == 0 and max(SZ) <= 128
    CPR = 2
    assert B % NW == 0
    RPW = B // NW
    RPQ = NBUF // CPR
    assert RPW % RPQ == 0
    NQ = RPW // RPQ

    mesh = plsc.VectorSubcoreMesh(core_axis_name="c", subcore_axis_name="s")

    @functools.partial(
        pl.kernel,
        out_type=jax.ShapeDtypeStruct((B * D,), jnp.float32),
        mesh=mesh,
        compiler_params=pltpu.CompilerParams(use_tc_tiling_on_sc=False),
        scratch_types=[
            pltpu.VMEM((RPW, T), jnp.int32),
            pltpu.VMEM((SZ[0], D), jnp.float32),
            pltpu.VMEM((SZ[1], D), jnp.float32),
            pltpu.VMEM((SZ[0], D), jnp.float32),
            pltpu.VMEM((SZ[1], D), jnp.float32),
            pltpu.VMEM((RPW * D,), jnp.float32),
            pltpu.SemaphoreType.DMA,
            pltpu.SemaphoreType.DMA,
            pltpu.SemaphoreType.DMA,
            pltpu.SemaphoreType.DMA,
        ],
    )
    def pool_kernel(ids_hbm, table_hbm, out_hbm,
                    idx_v, bu0, bu1, bu2, bu3, pooled_v, s0, s1, s2, s3):
        bufs = (bu0, bu1, bu2, bu3)
        sems = (s0, s1, s2, s3)
        wid = lax.axis_index("s") * NC + lax.axis_index("c")

        pltpu.sync_copy(ids_hbm.at[pl.ds(wid * RPW, RPW)], idx_v)

        def issue(row, c, b):
            pltpu.async_copy(
                table_hbm.at[idx_v.at[row, pl.ds(OFF[c], SZ[c])]],
                bufs[b], sems[b])

        def drain(c, b):
            pltpu.make_async_copy(
                table_hbm.at[pl.ds(0, SZ[c])], bufs[b], sems[b]).wait()

        def accum_chunk(buf, n, a0, a1):
            def step(i, carry):
                c0, c1 = carry
                t = i * 4
                p0 = (buf[t, pl.ds(0, L)] + buf[t + 1, pl.ds(0, L)]) + (
                    buf[t + 2, pl.ds(0, L)] + buf[t + 3, pl.ds(0, L)])
                p1 = (buf[t, pl.ds(L, L)] + buf[t + 1, pl.ds(L, L)]) + (
                    buf[t + 2, pl.ds(L, L)] + buf[t + 3, pl.ds(L, L)])
                return (c0 + p0, c1 + p1)
            return lax.fori_loop(0, n // 4, step, (a0, a1))

        for b in range(NBUF):
            issue(b // CPR, b % CPR, b)

        zero = jnp.zeros((L,), jnp.float32)

        def quad_body(q, _):
            for half in range(RPQ):
                row = q * RPQ + half
                a0, a1 = zero, zero
                for c in range(CPR):
                    b = half * CPR + c
                    drain(c, b)
                    a0, a1 = accum_chunk(bufs[b], SZ[c], a0, a1)

                    @pl.when(q < NQ - 1)
                    def _():
                        issue(row + RPQ, c, b)

                pooled_v[pl.ds(row * D, L)] = a0
                pooled_v[pl.ds(row * D + L, L)] = a1
            return 0

        lax.fori_loop(0, NQ, quad_body, 0)
        pltpu.sync_copy(pooled_v, out_hbm.at[pl.ds(wid * RPW * D, RPW * D)])

    return pool_kernel


@functools.lru_cache(maxsize=None)
def _build_classifier(B, D):
    BM = min(B, 512)
    assert B % BM == 0

    def body(p_ref, w_ref, b_ref, o_ref):
        o_ref[...] = jnp.dot(
            p_ref[...], w_ref[...],
            preferred_element_type=jnp.float32) + b_ref[...]

    return pl.pallas_call(
        body,
        grid=(B // BM,),
        in_specs=[
            pl.BlockSpec((BM, D), lambda i: (i, 0)),
            pl.BlockSpec((D, NPAD), lambda i: (0, 0)),
            pl.BlockSpec((1, NPAD), lambda i: (0, 0)),
        ],
        out_specs=pl.BlockSpec((BM, NPAD), lambda i: (i, 0)),
        out_shape=jax.ShapeDtypeStruct((B, NPAD), jnp.float32),
    )


@jax.jit
def kernel(input_ids, emb_table, W, b):
    B, T = input_ids.shape
    V, D = emb_table.shape
    pooled = _build_pool(B, T, V, D)(
        input_ids.astype(jnp.int32), emb_table).reshape(B, D)
    wpad = jnp.zeros((D, NPAD), jnp.float32)
    wpad = wpad.at[:, :NOUT].set(W.astype(jnp.float32) * (1.0 / T))
    bpad = jnp.zeros((1, NPAD), jnp.float32).at[0, :NOUT].set(
        b.astype(jnp.float32))
    logits = _build_classifier(B, D)(pooled, wpad, bpad)
    return logits[:, :NOUT]
